# baseline (speedup 1.0000x reference)
import functools

import jax
import jax.numpy as jnp
from jax import lax
from jax.experimental import pallas as pl
from jax.experimental.pallas import tpu as pltpu
from jax.experimental.pallas import tpu_sc as plsc

NC = 2
NS = 16
NW = NC * NS
L = 16

F = 60
D = 32
G = 128
MB = 2
IPM = MB * F


@functools.partial(jax.jit, static_argnums=(4, 5))
def _fm_sc(xflat, etab4, wflat, bias16, B, RPW):
  NMB = RPW // MB

  mesh = plsc.VectorSubcoreMesh(core_axis_name="c", subcore_axis_name="s")

  @functools.partial(
      pl.kernel,
      out_type=jax.ShapeDtypeStruct((B,), jnp.float32),
      mesh=mesh,
      scratch_types=[
          pltpu.VMEM((RPW * F,), jnp.int32),
          pltpu.VMEM((IPM, G), jnp.float32),
          pltpu.VMEM((IPM, G), jnp.float32),
          pltpu.VMEM((IPM + L,), jnp.float32),
          pltpu.VMEM((IPM + L,), jnp.float32),
          pltpu.VMEM((IPM,), jnp.int32),
          pltpu.VMEM((IPM,), jnp.int32),
          pltpu.VMEM((RPW,), jnp.float32),
          pltpu.VMEM((L,), jnp.float32),
          pltpu.SemaphoreType.DMA,
          pltpu.SemaphoreType.DMA,
      ],
      compiler_params=pltpu.CompilerParams(use_tc_tiling_on_sc=True),
  )
  def body(x_hbm, tab_hbm, w_hbm, bias_hbm, out_hbm,
           idx_v, ebuf0, ebuf1, wbuf0, wbuf1, gidx0, gidx1,
           obuf, bias_v, sem0, sem1):
    wid = lax.axis_index("s") * NC + lax.axis_index("c")
    base = wid * RPW
    pltpu.sync_copy(x_hbm.at[pl.ds(base * F, RPW * F)], idx_v)
    pltpu.sync_copy(bias_hbm, bias_v)

    lane = lax.iota(jnp.int32, L)
    tailmask = lane < (F - 3 * L)
    zero16 = jnp.zeros((L,), jnp.float32)
    NV = IPM // L

    def issue(m, ebuf, wbuf, gidx, sem):
      off = m * IPM
      for t in range(8):
        o = min(t * L, IPM - L)
        gidx[pl.ds(o, L)] = lax.shift_right_logical(
            idx_v[pl.ds(off + o, L)], 2)
      pltpu.async_copy(tab_hbm.at[gidx], ebuf, sem)
      pltpu.async_copy(w_hbm.at[idx_v.at[pl.ds(off, IPM)]],
                       wbuf.at[pl.ds(0, IPM)], sem)

    def drain(ebuf, wbuf, sem):
      pltpu.make_async_copy(tab_hbm.at[pl.ds(0, IPM)], ebuf, sem).wait()
      pltpu.make_async_copy(w_hbm.at[pl.ds(0, IPM)], wbuf.at[pl.ds(0, IPM)], sem).wait()

    def compute(m, ebuf, wbuf, y):
      off = m * IPM
      for r in range(MB):
        rb = r * F
        iv = [idx_v[pl.ds(off + rb + min(t * L, F - L), L)] for t in range(4)]
        s0 = zero16
        s1 = zero16
        q0 = zero16
        q1 = zero16
        for j in range(F):
          t, l = divmod(j, L)
          if t == 3:
            t, l = 3, j - (F - L)
          v_s = iv[t][l]
          cb = (v_s & 3) * D
          x0 = ebuf[rb + j, pl.ds(cb, L)]
          x1 = ebuf[rb + j, pl.ds(cb + L, L)]
          s0 = s0 + x0
          s1 = s1 + x1
          q0 = q0 + x0 * x0
          q1 = q1 + x1 * x1
        wv = (wbuf[pl.ds(rb, L)] + wbuf[pl.ds(rb + L, L)]
              + wbuf[pl.ds(rb + 2 * L, L)]
              + jnp.where(tailmask, wbuf[pl.ds(rb + 3 * L, L)], 0.0))
        u = wv + 0.5 * (s0 * s0 - q0 + s1 * s1 - q1)
        z = u[0]
        for i in range(1, L):
          z = z + u[i]
        y = jnp.where(lane == (m * MB + r) % L, z, y)
      return y

    issue(0, ebuf0, wbuf0, gidx0, sem0)

    def mb_pair(k, y):
      m0 = 2 * k
      m1 = 2 * k + 1
      issue(m1, ebuf1, wbuf1, gidx1, sem1)
      drain(ebuf0, wbuf0, sem0)
      y = compute(m0, ebuf0, wbuf0, y)

      @pl.when(m1 + 1 < NMB)
      def _():
        issue(m1 + 1, ebuf0, wbuf0, gidx0, sem0)

      drain(ebuf1, wbuf1, sem1)
      y = compute(m1, ebuf1, wbuf1, y)

      @pl.when(k % 4 == 3)
      def _():
        yv = y + bias_v[...]
        obuf[pl.ds((k // 4) * L, L)] = 1.0 / (1.0 + jnp.exp(-yv))

      return jnp.where(k % 4 == 3, zero16, y)

    lax.fori_loop(0, NMB // 2, mb_pair, zero16)
    pltpu.sync_copy(obuf, out_hbm.at[pl.ds(base, RPW)])

  return body(xflat, etab4, wflat, bias16)


def kernel(item_tag1, item_tag2, item_tag3, embed_table, w_table, bias):
  B = item_tag1.shape[0]
  X = jnp.concatenate([item_tag1, item_tag2, item_tag3], axis=1)
  xflat = X.reshape(-1).astype(jnp.int32)
  wflat = w_table.reshape(-1).astype(jnp.float32)
  bias16 = jnp.broadcast_to(bias.astype(jnp.float32), (L,))
  etab4 = embed_table.reshape(embed_table.shape[0] // 4, 4 * D)
  out = _fm_sc(xflat, etab4, wflat, bias16, B, B // NW)
  return out.reshape(B, 1)


# speedup vs baseline: 1.3343x; 1.3343x over previous
import functools

import jax
import jax.numpy as jnp
from jax import lax
from jax.experimental import pallas as pl
from jax.experimental.pallas import tpu as pltpu
from jax.experimental.pallas import tpu_sc as plsc

NC = 2
NS = 16
NW = NC * NS
L = 16

F = 60
D = 32
MB = 16
CH = 120
IPM = MB * F
NCH = IPM // CH


@functools.partial(jax.jit, static_argnums=(4, 5))
def _fm_sc(xflat, embed_table, wflat, bias16, B, RPW):
  NMB = RPW // MB

  mesh = plsc.VectorSubcoreMesh(core_axis_name="c", subcore_axis_name="s")

  @functools.partial(
      pl.kernel,
      out_type=jax.ShapeDtypeStruct((B,), jnp.float32),
      mesh=mesh,
      scratch_types=[
          pltpu.VMEM((RPW * F,), jnp.int32),
          pltpu.VMEM((IPM, D), jnp.float32),
          pltpu.VMEM((IPM, D), jnp.float32),
          pltpu.VMEM((IPM + L,), jnp.float32),
          pltpu.VMEM((IPM + L,), jnp.float32),
          pltpu.VMEM((RPW,), jnp.float32),
          pltpu.VMEM((L,), jnp.float32),
          pltpu.SemaphoreType.DMA,
          pltpu.SemaphoreType.DMA,
      ],
      compiler_params=pltpu.CompilerParams(use_tc_tiling_on_sc=False),
  )
  def body(x_hbm, tab_hbm, w_hbm, bias_hbm, out_hbm,
           idx_v, ebuf0, ebuf1, wbuf0, wbuf1, obuf, bias_v, sem0, sem1):
    wid = lax.axis_index("s") * NC + lax.axis_index("c")
    base = wid * RPW
    pltpu.sync_copy(x_hbm.at[pl.ds(base * F, RPW * F)], idx_v)
    pltpu.sync_copy(bias_hbm, bias_v)

    lane = lax.iota(jnp.int32, L)
    tailmask = lane < (F - 3 * L)
    zero16 = jnp.zeros((L,), jnp.float32)

    def issue(m, ebuf, wbuf, sem):
      off = m * IPM
      for c in range(NCH):
        ii = idx_v.at[pl.ds(off + c * CH, CH)]
        pltpu.async_copy(tab_hbm.at[ii], ebuf.at[pl.ds(c * CH, CH)], sem)
        pltpu.async_copy(w_hbm.at[ii], wbuf.at[pl.ds(c * CH, CH)], sem)

    def drain(ebuf, wbuf, sem):
      pltpu.make_async_copy(tab_hbm.at[pl.ds(0, IPM)], ebuf, sem).wait()
      pltpu.make_async_copy(w_hbm.at[pl.ds(0, IPM)], wbuf.at[pl.ds(0, IPM)], sem).wait()

    def compute(m, ebuf, wbuf):
      def row_body(r, y):
        s0 = zero16
        s1 = zero16
        q0 = zero16
        q1 = zero16
        rb = r * F
        for j in range(F):
          x0 = ebuf[rb + j, pl.ds(0, L)]
          x1 = ebuf[rb + j, pl.ds(L, L)]
          s0 = s0 + x0
          s1 = s1 + x1
          q0 = q0 + x0 * x0
          q1 = q1 + x1 * x1
        wv = (wbuf[pl.ds(rb, L)] + wbuf[pl.ds(rb + L, L)]
              + wbuf[pl.ds(rb + 2 * L, L)]
              + jnp.where(tailmask, wbuf[pl.ds(rb + 3 * L, L)], 0.0))
        u = wv + 0.5 * (s0 * s0 - q0 + s1 * s1 - q1)
        z = u[0]
        for i in range(1, L):
          z = z + u[i]
        return jnp.where(lane == r, z, y)

      y = lax.fori_loop(0, MB, row_body, zero16) + bias_v[...]
      y = 1.0 / (1.0 + jnp.exp(-y))
      obuf[pl.ds(m * MB, MB)] = y

    issue(0, ebuf0, wbuf0, sem0)

    def mb_pair(k, carry):
      m0 = 2 * k
      m1 = 2 * k + 1
      issue(m1, ebuf1, wbuf1, sem1)
      drain(ebuf0, wbuf0, sem0)
      compute(m0, ebuf0, wbuf0)

      @pl.when(m1 + 1 < NMB)
      def _():
        issue(m1 + 1, ebuf0, wbuf0, sem0)

      drain(ebuf1, wbuf1, sem1)
      compute(m1, ebuf1, wbuf1)
      return carry

    lax.fori_loop(0, NMB // 2, mb_pair, 0)
    pltpu.sync_copy(obuf, out_hbm.at[pl.ds(base, RPW)])

  return body(xflat, embed_table, wflat, bias16)


def kernel(item_tag1, item_tag2, item_tag3, embed_table, w_table, bias):
  B = item_tag1.shape[0]
  X = jnp.concatenate([item_tag1, item_tag2, item_tag3], axis=1)
  xflat = X.reshape(-1).astype(jnp.int32)
  wflat = w_table.reshape(-1).astype(jnp.float32)
  bias16 = jnp.broadcast_to(bias.astype(jnp.float32), (L,))
  out = _fm_sc(xflat, embed_table, wflat, bias16, B, B // NW)
  return out.reshape(B, 1)
